# baseline (speedup 1.0000x reference)
import functools

import jax
import jax.numpy as jnp
from jax import lax
from jax.experimental import pallas as pl
from jax.experimental.pallas import tpu as pltpu
from jax.experimental.pallas import tpu_sc as plsc

D = 32
BB = 128
L = 16
TP = BB + 1


def _build(B, H):
    N = B * H
    info = plsc.get_sparse_core_info()
    NC, NS = info.num_cores, info.num_subcores
    NW = NC * NS
    n_units = N // BB
    u_per_w = n_units // NW
    assert n_units % NW == 0 and B % BB == 0
    idx_per_w = N // NW
    NBUF = 4
    n_steps = u_per_w // NBUF
    BT = B // BB
    CT = D // 8

    mesh = plsc.VectorSubcoreMesh(core_axis_name="c", subcore_axis_name="s")

    @functools.partial(
        pl.kernel,
        mesh=mesh,
        out_type=jax.ShapeDtypeStruct((H, CT, BT, 8, BB), jnp.float32),
        compiler_params=pltpu.CompilerParams(
            use_tc_tiling_on_sc=False, needs_layout_passes=False
        ),
        scratch_types=[
            pltpu.VMEM((idx_per_w,), jnp.int32),
            pltpu.VMEM((NBUF, BB, D), jnp.float32),
            pltpu.VMEM((NBUF, CT, 8, TP), jnp.float32),
        ]
        + [pltpu.SemaphoreType.DMA] * (2 * NBUF),
    )
    def k(idx_hbm, table_hbm, out_hbm, idx_v, rows_v, tr_v, *sems):
        gsem = sems[:NBUF]
        osem = sems[NBUF:]
        wid = lax.axis_index("s") * NC + lax.axis_index("c")
        base = wid * idx_per_w
        u0 = wid * u_per_w
        pltpu.sync_copy(idx_hbm.at[pl.ds(base, idx_per_w)], idx_v)

        lane = lax.iota(jnp.int32, L)
        ct_vec = [lane // 8, lane // 8 + 2]
        cs_vec = lane % 8

        def gather_copy(ul, b):
            off = pl.multiple_of(ul * BB, BB)
            return pltpu.make_async_copy(
                table_hbm.at[idx_v.at[pl.ds(off, BB)]], rows_v.at[b], gsem[b]
            )

        def store_copy(ul, b):
            u = u0 + ul
            h = lax.div(u, BT)
            bt = lax.rem(u, BT)
            return pltpu.make_async_copy(
                tr_v.at[b, :, :, pl.ds(0, BB)], out_hbm.at[h, :, bt], osem[b]
            )

        def transpose(b):
            dst = tr_v.at[b]
            for j in range(BB):
                jv = jnp.full((L,), j, jnp.int32)
                v0 = rows_v[b, j, pl.ds(0, L)]
                v1 = rows_v[b, j, pl.ds(L, L)]
                plsc.store_scatter(dst, [ct_vec[0], cs_vec, jv], v0)
                plsc.store_scatter(dst, [ct_vec[1], cs_vec, jv], v1)

        for b in range(NBUF):
            gather_copy(b, b).start()

        for b in range(NBUF):
            gather_copy(b, b).wait()
            transpose(b)
            gather_copy(b + NBUF, b).start()
            store_copy(b, b).start()

        def body(s, carry):
            for b in range(NBUF):
                ul = s * NBUF + b
                gather_copy(ul, b).wait()
                store_copy(ul - NBUF, b).wait()
                transpose(b)
                gather_copy(ul + NBUF, b).start()
                store_copy(ul, b).start()
            return carry

        lax.fori_loop(1, n_steps - 1, body, 0)

        for b in range(NBUF):
            ul = (n_steps - 1) * NBUF + b
            gather_copy(ul, b).wait()
            store_copy(ul - NBUF, b).wait()
            transpose(b)
            store_copy(ul, b).start()
        for b in range(NBUF):
            ul = (n_steps - 1) * NBUF + b
            store_copy(ul, b).wait()

    return k


def kernel(token_ids, emb):
    B, H = token_ids.shape
    idx = token_ids.T.reshape(B * H).astype(jnp.int32)
    k = _build(B, H)
    out5 = k(idx, emb)
    return out5.transpose(2, 4, 0, 1, 3).reshape(B, H, D)


# speedup vs baseline: 1.1145x; 1.1145x over previous
import functools

import jax
import jax.numpy as jnp
from jax import lax
from jax.experimental import pallas as pl
from jax.experimental.pallas import tpu as pltpu
from jax.experimental.pallas import tpu_sc as plsc

D = 32
BB = 128
L = 16
TP = BB + 1


def _build(B, H):
    N = B * H
    info = plsc.get_sparse_core_info()
    NC, NS = info.num_cores, info.num_subcores
    NW = NC * NS
    n_units = N // BB
    u_per_w = n_units // NW
    assert n_units % NW == 0 and B % BB == 0
    idx_per_w = N // NW
    NBUF = 2
    n_steps = u_per_w // NBUF
    BT = B // BB
    CT = D // 8

    mesh = plsc.VectorSubcoreMesh(core_axis_name="c", subcore_axis_name="s")

    @functools.partial(
        pl.kernel,
        mesh=mesh,
        out_type=jax.ShapeDtypeStruct((H, CT, BT, 8, BB), jnp.float32),
        compiler_params=pltpu.CompilerParams(
            use_tc_tiling_on_sc=False, needs_layout_passes=False
        ),
        scratch_types=[
            pltpu.VMEM((idx_per_w,), jnp.int32),
            pltpu.VMEM((NBUF, BB, D), jnp.float32),
            pltpu.VMEM((NBUF, CT, 8, TP), jnp.float32),
        ]
        + [pltpu.SemaphoreType.DMA] * (2 * NBUF),
    )
    def k(idx_hbm, table_hbm, out_hbm, idx_v, rows_v, tr_v, *sems):
        gsem = sems[:NBUF]
        osem = sems[NBUF:]
        wid = lax.axis_index("s") * NC + lax.axis_index("c")
        base = wid * idx_per_w
        u0 = wid * u_per_w
        pltpu.sync_copy(idx_hbm.at[pl.ds(base, idx_per_w)], idx_v)

        lane = lax.iota(jnp.int32, L)
        ct_vec = [lane // 8, lane // 8 + 2]
        cs_vec = lane % 8

        def gather_copy(ul, b):
            off = pl.multiple_of(ul * BB, BB)
            return pltpu.make_async_copy(
                table_hbm.at[idx_v.at[pl.ds(off, BB)]], rows_v.at[b], gsem[b]
            )

        def store_copy(ul, b):
            u = u0 + ul
            h = lax.div(u, BT)
            bt = lax.rem(u, BT)
            return pltpu.make_async_copy(
                tr_v.at[b, :, :, pl.ds(0, BB)], out_hbm.at[h, :, bt], osem[b]
            )

        def transpose(b):
            dst = tr_v.at[b]
            for j in range(BB):
                jv = jnp.full((L,), j, jnp.int32)
                v0 = rows_v[b, j, pl.ds(0, L)]
                v1 = rows_v[b, j, pl.ds(L, L)]
                plsc.store_scatter(dst, [ct_vec[0], cs_vec, jv], v0)
                plsc.store_scatter(dst, [ct_vec[1], cs_vec, jv], v1)

        for b in range(NBUF):
            gather_copy(b, b).start()

        for b in range(NBUF):
            gather_copy(b, b).wait()
            transpose(b)
            gather_copy(b + NBUF, b).start()
            store_copy(b, b).start()

        def body(s, carry):
            for b in range(NBUF):
                ul = s * NBUF + b
                gather_copy(ul, b).wait()
                store_copy(ul - NBUF, b).wait()
                transpose(b)
                gather_copy(ul + NBUF, b).start()
                store_copy(ul, b).start()
            return carry

        lax.fori_loop(1, n_steps - 1, body, 0)

        for b in range(NBUF):
            ul = (n_steps - 1) * NBUF + b
            gather_copy(ul, b).wait()
            store_copy(ul - NBUF, b).wait()
            transpose(b)
            store_copy(ul, b).start()
        for b in range(NBUF):
            ul = (n_steps - 1) * NBUF + b
            store_copy(ul, b).wait()

    return k


def kernel(token_ids, emb):
    B, H = token_ids.shape
    idx = token_ids.T.reshape(B * H).astype(jnp.int32)
    k = _build(B, H)
    out5 = k(idx, emb)
    return out5.transpose(2, 4, 0, 1, 3).reshape(B, H, D)
